# baseline (speedup 1.0000x reference)
import jax
import jax.numpy as jnp
from jax.experimental import pallas as pl
from jax.experimental.pallas import tpu as pltpu

_CHUNK = 4096


def _loss_kernel(pref, kp, out):
    B = kp.shape[0]
    N4 = kp.shape[1]
    n_chunks = N4 // _CHUNK

    P = [[[pref[:, 16 * i + 2 * j + c:16 * i + 2 * j + c + 1]
           for c in range(2)]
          for j in range(3)] for i in range(3)]

    lane4 = jax.lax.broadcasted_iota(jnp.int32, (B, _CHUNK), 1) % 4
    is_src_y = lane4 == 0

    acc = jnp.zeros((B, _CHUNK), jnp.float32)
    for ci in range(n_chunks):
        v = kp[:, pl.ds(ci * _CHUNK, _CHUNK)]

        pn = v / 255.5 - 1.0
        t = (pn + 1.0) * 0.5 * 511.0

        t0 = jnp.floor(t)
        f = t - t0
        w0 = 1.0 - f

        zero = jnp.zeros_like(t)
        p0 = (jnp.where(t0 == 0.0, w0, zero)
              + jnp.where(t0 == -1.0, f, zero))
        p1 = (jnp.where(t0 == 1.0, w0, zero)
              + jnp.where(t0 == 0.0, f, zero))
        p2 = jnp.where(t0 == 1.0, f, zero)

        px0 = pltpu.roll(p0, _CHUNK - 2, 1)
        px1 = pltpu.roll(p1, _CHUNK - 2, 1)
        px2 = pltpu.roll(p2, _CHUNK - 2, 1)
        ty = pltpu.roll(v, _CHUNK - 1, 1)
        tx = pltpu.roll(v, _CHUNK - 3, 1)

        pys = (p0, p1, p2)
        pxs = (px0, px1, px2)
        loc0 = zero
        loc1 = zero
        for i in range(3):
            for j in range(3):
                w = pys[i] * pxs[j]
                loc0 = loc0 + P[i][j][0] * w
                loc1 = loc1 + P[i][j][1] * w

        d0 = loc0 - ty + 1e-6
        d1 = loc1 - tx + 1e-6
        dist = jnp.sqrt(d0 * d0 + d1 * d1)
        acc = acc + jnp.where(is_src_y, dist, zero)

    s = jnp.sum(acc, axis=(0, 1), keepdims=True)
    out[:, :] = s / (B * N4 // 4)


def kernel(kp_preds, kp_pairs):
    B, H, W, C = kp_preds.shape
    N = kp_pairs.shape[1]
    patch = jax.lax.slice(kp_preds, (0, 0, 0, 0), (B, 4, 8, 2))
    pref = patch.reshape(B, 64)
    kp = (kp_pairs + 0.0).reshape(B, N * 4)
    out = pl.pallas_call(
        _loss_kernel,
        grid=(1,),
        in_specs=[
            pl.BlockSpec((B, 64), lambda i: (0, 0)),
            pl.BlockSpec((B, N * 4), lambda i: (0, 0)),
        ],
        out_specs=pl.BlockSpec((1, 1), lambda i: (0, 0)),
        out_shape=jax.ShapeDtypeStruct((1, 1), jnp.float32),
    )(pref, kp)
    return out[0, 0]
